# Initial kernel scaffold; baseline (speedup 1.0000x reference)
import functools

import jax
import jax.numpy as jnp
from jax import lax
from jax.experimental import pallas as pl
from jax.experimental.pallas import tpu as pltpu
from jax.experimental.pallas import tpu_sc as plsc

B = 4096
HIST = 50
D = 64
NCOLS = 6
ELEM = D * NCOLS

_INFO = plsc.get_sparse_core_info()
_NC = _INFO.num_cores
_NS = _INFO.num_subcores
_NW = _NC * _NS
_BPW = B // _NW
_CHUNK = 128
_NCHUNK = (_BPW * HIST) // _CHUNK

_sc_mesh = plsc.VectorSubcoreMesh(core_axis_name="c", subcore_axis_name="s")


@functools.partial(
    pl.kernel,
    out_type=[jax.ShapeDtypeStruct((B, D), jnp.float32) for _ in range(6)],
    mesh=_sc_mesh,
    scratch_types=[
        pltpu.VMEM((_BPW,), jnp.int32),
        pltpu.VMEM((_BPW, D), jnp.float32),
        pltpu.VMEM((_NCHUNK, _CHUNK), jnp.int32),
        pltpu.VMEM((_NCHUNK, _CHUNK), jnp.int32),
        pltpu.VMEM((_CHUNK, D), jnp.float32),
        pltpu.VMEM((_BPW, D), jnp.float32),
        pltpu.SemaphoreType.DMA,
        pltpu.SemaphoreType.DMA,
    ],
)
def _sc_gather(idx_user, idx_item, idx_cate, idx_hour, idx_device,
               hist_flat, seg_hbm, zeros_hbm,
               tab_user, tab_item, tab_cate, tab_hour, tab_device, tab_hist,
               out_user, out_item, out_cate, out_hour, out_device, out_hist,
               idx_v, rows_v, hidx_v, seg_v, hrows_v, acc_v, sem, sem2):
    wid = lax.axis_index("s") * _NC + lax.axis_index("c")
    base = wid * _BPW

    for idx_hbm, tab_hbm, out_hbm in (
        (idx_user, tab_user, out_user),
        (idx_item, tab_item, out_item),
        (idx_cate, tab_cate, out_cate),
        (idx_hour, tab_hour, out_hour),
        (idx_device, tab_device, out_device),
    ):
        pltpu.sync_copy(idx_hbm.at[pl.ds(base, _BPW)], idx_v)
        pltpu.async_copy(tab_hbm.at[idx_v], rows_v, sem).wait()
        pltpu.sync_copy(rows_v, out_hbm.at[pl.ds(base, _BPW)])

    pltpu.sync_copy(
        hist_flat.at[pl.ds(base * HIST, _BPW * HIST)],
        hidx_v.reshape(_NCHUNK * _CHUNK),
    )
    pltpu.sync_copy(seg_hbm, seg_v)
    pltpu.sync_copy(zeros_hbm, acc_v)

    def chunk_body(j, carry):
        pltpu.async_copy(tab_hist.at[hidx_v.at[j]], hrows_v, sem).wait()
        pltpu.async_copy(hrows_v, acc_v.at[seg_v.at[j]], sem2, add=True).wait()
        return carry

    lax.fori_loop(0, _NCHUNK, chunk_body, 0)
    pltpu.sync_copy(acc_v, out_hist.at[pl.ds(base, _BPW)])


def _mlp_body(eu, ei, ec, eh, ed, ehist, w1, b1, w2, b2, w3, b3, out):
    x = jnp.concatenate(
        [eu[...], ei[...], ec[...], eh[...], ed[...],
         ehist[...] * (1.0 / HIST)], axis=1)
    h = jax.nn.relu(jnp.dot(x, w1[...], preferred_element_type=jnp.float32)
                    + b1[...])
    h = jax.nn.relu(jnp.dot(h, w2[...], preferred_element_type=jnp.float32)
                    + b2[...])
    out[...] = jax.nn.sigmoid(
        jnp.dot(h, w3[...], preferred_element_type=jnp.float32) + b3[...])


def _mlp(feats, W1, b1, W2, b2, W3, b3):
    BB = 512
    grid = (B // BB,)
    feat_spec = pl.BlockSpec((BB, D), lambda i: (i, 0))
    full = lambda shape: pl.BlockSpec(shape, lambda i: tuple(0 for _ in shape))
    return pl.pallas_call(
        _mlp_body,
        grid=grid,
        in_specs=[feat_spec] * 6 + [
            full((ELEM, ELEM)), full((ELEM,)),
            full((ELEM, ELEM)), full((ELEM,)),
            full((ELEM, 1)), full((1,)),
        ],
        out_specs=pl.BlockSpec((BB, 1), lambda i: (i, 0)),
        out_shape=jax.ShapeDtypeStruct((B, 1), jnp.float32),
    )(*feats, W1, b1, W2, b2, W3, b3)


def kernel(idx_user, idx_item, idx_cate, idx_hour, idx_device, idx_hist,
           tab_user, tab_item, tab_cate, tab_hour, tab_device, tab_hist,
           W1, b1, W2, b2, W3, b3):
    i32 = lambda a: a.astype(jnp.int32)
    hist_flat = i32(idx_hist).reshape(B * HIST)
    seg = (jnp.arange(_BPW * HIST, dtype=jnp.int32) // HIST).reshape(
        _NCHUNK, _CHUNK)
    zeros = jnp.zeros((_BPW, D), jnp.float32)
    feats = _sc_gather(
        i32(idx_user), i32(idx_item), i32(idx_cate), i32(idx_hour),
        i32(idx_device), hist_flat, seg, zeros,
        tab_user, tab_item, tab_cate, tab_hour, tab_device, tab_hist)
    return _mlp(feats, W1, b1, W2, b2, W3, b3)


# speedup vs baseline: 3.6226x; 3.6226x over previous
import functools

import jax
import jax.numpy as jnp
from jax import lax
from jax.experimental import pallas as pl
from jax.experimental.pallas import tpu as pltpu
from jax.experimental.pallas import tpu_sc as plsc

B = 4096
HIST = 50
D = 64
NCOLS = 6
ELEM = D * NCOLS

_INFO = plsc.get_sparse_core_info()
_NC = _INFO.num_cores
_NS = _INFO.num_subcores
_NW = _NC * _NS
_BPW = B // _NW
_CHUNK = 128
_NCHUNK = (_BPW * HIST) // _CHUNK

_sc_mesh = plsc.VectorSubcoreMesh(core_axis_name="c", subcore_axis_name="s")


@functools.partial(
    pl.kernel,
    out_type=[jax.ShapeDtypeStruct((B, D), jnp.float32) for _ in range(6)],
    mesh=_sc_mesh,
    scratch_types=[
        pltpu.VMEM((_BPW,), jnp.int32),
        pltpu.VMEM((_BPW, D), jnp.float32),
        pltpu.VMEM((_NCHUNK, _CHUNK), jnp.int32),
        pltpu.VMEM((_NCHUNK, _CHUNK), jnp.int32),
        pltpu.VMEM((_CHUNK, D), jnp.float32),
        pltpu.VMEM_SHARED((_NS * _BPW, D), jnp.float32),
        pltpu.SemaphoreType.DMA,
        pltpu.SemaphoreType.DMA,
    ],
    compiler_params=pltpu.CompilerParams(use_tc_tiling_on_sc=False),
)
def _sc_gather(idx_user, idx_item, idx_cate, idx_hour, idx_device,
               hist_flat, seg_hbm, zeros_hbm,
               tab_user, tab_item, tab_cate, tab_hour, tab_device, tab_hist,
               out_user, out_item, out_cate, out_hour, out_device, out_hist,
               idx_v, rows_v, hidx_v, seg_v, hrows_v, acc_sh, sem, sem2):
    sid = lax.axis_index("s")
    wid = sid * _NC + lax.axis_index("c")
    base = wid * _BPW

    for idx_hbm, tab_hbm, out_hbm in (
        (idx_user, tab_user, out_user),
        (idx_item, tab_item, out_item),
        (idx_cate, tab_cate, out_cate),
        (idx_hour, tab_hour, out_hour),
        (idx_device, tab_device, out_device),
    ):
        pltpu.sync_copy(idx_hbm.at[pl.ds(base, _BPW)], idx_v)
        pltpu.async_copy(tab_hbm.at[idx_v], rows_v, sem).wait()
        pltpu.sync_copy(rows_v, out_hbm.at[pl.ds(base, _BPW)])

    pltpu.sync_copy(hist_flat.at[wid], hidx_v)
    pltpu.sync_copy(seg_hbm.at[sid], seg_v)
    pltpu.sync_copy(zeros_hbm, acc_sh.at[pl.ds(sid * _BPW, _BPW)])

    def chunk_body(j, carry):
        pltpu.async_copy(tab_hist.at[hidx_v.at[j]], hrows_v, sem).wait()
        pltpu.async_copy(hrows_v, acc_sh.at[seg_v.at[j]], sem2,
                         add=True).wait()
        return carry

    lax.fori_loop(0, _NCHUNK, chunk_body, 0)
    pltpu.sync_copy(acc_sh.at[pl.ds(sid * _BPW, _BPW)],
                    out_hist.at[pl.ds(base, _BPW)])


def _mlp_body(eu, ei, ec, eh, ed, ehist, w1, b1, w2, b2, w3, b3, out):
    x = jnp.concatenate(
        [eu[...], ei[...], ec[...], eh[...], ed[...],
         ehist[...] * (1.0 / HIST)], axis=1)
    h = jax.nn.relu(jnp.dot(x, w1[...], preferred_element_type=jnp.float32)
                    + b1[...])
    h = jax.nn.relu(jnp.dot(h, w2[...], preferred_element_type=jnp.float32)
                    + b2[...])
    out[...] = jax.nn.sigmoid(
        jnp.dot(h, w3[...], preferred_element_type=jnp.float32) + b3[...])


def _mlp(feats, W1, b1, W2, b2, W3, b3):
    BB = 512
    grid = (B // BB,)
    feat_spec = pl.BlockSpec((BB, D), lambda i: (i, 0))
    full = lambda shape: pl.BlockSpec(shape, lambda i: tuple(0 for _ in shape))
    return pl.pallas_call(
        _mlp_body,
        grid=grid,
        in_specs=[feat_spec] * 6 + [
            full((ELEM, ELEM)), full((1, ELEM)),
            full((ELEM, ELEM)), full((1, ELEM)),
            full((ELEM, 1)), full((1, 1)),
        ],
        out_specs=pl.BlockSpec((BB, 1), lambda i: (i, 0)),
        out_shape=jax.ShapeDtypeStruct((B, 1), jnp.float32),
    )(*feats, W1, b1.reshape(1, ELEM), W2, b2.reshape(1, ELEM),
      W3, b3.reshape(1, 1))


def kernel(idx_user, idx_item, idx_cate, idx_hour, idx_device, idx_hist,
           tab_user, tab_item, tab_cate, tab_hour, tab_device, tab_hist,
           W1, b1, W2, b2, W3, b3):
    i32 = lambda a: a.astype(jnp.int32)
    hist_flat = i32(idx_hist).reshape(_NW, _NCHUNK, _CHUNK)
    seg_local = (jnp.arange(_BPW * HIST, dtype=jnp.int32) // HIST).reshape(
        1, _NCHUNK, _CHUNK)
    seg = seg_local + (jnp.arange(_NS, dtype=jnp.int32) * _BPW).reshape(
        _NS, 1, 1)
    zeros = jnp.zeros((_BPW, D), jnp.float32)
    feats = _sc_gather(
        i32(idx_user), i32(idx_item), i32(idx_cate), i32(idx_hour),
        i32(idx_device), hist_flat, seg, zeros,
        tab_user, tab_item, tab_cate, tab_hour, tab_device, tab_hist)
    return _mlp(feats, W1, b1, W2, b2, W3, b3)


# speedup vs baseline: 3.9302x; 1.0849x over previous
import functools

import jax
import jax.numpy as jnp
from jax import lax
from jax.experimental import pallas as pl
from jax.experimental.pallas import tpu as pltpu
from jax.experimental.pallas import tpu_sc as plsc

B = 4096
HIST = 50
D = 64
NCOLS = 6
ELEM = D * NCOLS

_INFO = plsc.get_sparse_core_info()
_NC = _INFO.num_cores
_NS = _INFO.num_subcores
_NW = _NC * _NS
_BPW = B // _NW
_CHUNK = 128
_NCHUNK = (_BPW * HIST) // _CHUNK

_sc_mesh = plsc.VectorSubcoreMesh(core_axis_name="c", subcore_axis_name="s")


@functools.partial(
    pl.kernel,
    out_type=[jax.ShapeDtypeStruct((B, D), jnp.float32) for _ in range(6)],
    mesh=_sc_mesh,
    scratch_types=[
        pltpu.VMEM((5, _BPW), jnp.int32),
        pltpu.VMEM((5, _BPW, D), jnp.float32),
        pltpu.VMEM((_NCHUNK, _CHUNK), jnp.int32),
        pltpu.VMEM((_NCHUNK, _CHUNK), jnp.int32),
        pltpu.VMEM((2, _CHUNK, D), jnp.float32),
        pltpu.VMEM_SHARED((_NS * _BPW, D), jnp.float32),
        pltpu.SemaphoreType.DMA,
        pltpu.SemaphoreType.DMA,
        pltpu.SemaphoreType.DMA,
        pltpu.SemaphoreType.DMA,
        pltpu.SemaphoreType.DMA,
        pltpu.SemaphoreType.DMA,
        pltpu.SemaphoreType.DMA,
    ],
    compiler_params=pltpu.CompilerParams(use_tc_tiling_on_sc=False),
)
def _sc_gather(idx_user, idx_item, idx_cate, idx_hour, idx_device,
               hist_flat, seg_hbm, zeros_hbm,
               tab_user, tab_item, tab_cate, tab_hour, tab_device, tab_hist,
               out_user, out_item, out_cate, out_hour, out_device, out_hist,
               fidx_v, frows_v, hidx_v, seg_v, hbuf_v, acc_sh,
               sem_m, sem_f, sem_g0, sem_g1, sem_s0, sem_s1, sem_o):
    sid = lax.axis_index("s")
    wid = sid * _NC + lax.axis_index("c")
    base = wid * _BPW
    acc_slot = acc_sh.at[pl.ds(sid * _BPW, _BPW)]
    sem_g = (sem_g0, sem_g1)
    sem_s = (sem_s0, sem_s1)

    feats = ((idx_user, tab_user, out_user),
             (idx_item, tab_item, out_item),
             (idx_cate, tab_cate, out_cate),
             (idx_hour, tab_hour, out_hour),
             (idx_device, tab_device, out_device))

    stage = [pltpu.async_copy(idx_hbm.at[pl.ds(base, _BPW)], fidx_v.at[k],
                              sem_m)
             for k, (idx_hbm, _, _) in enumerate(feats)]
    stage.append(pltpu.async_copy(hist_flat.at[wid], hidx_v, sem_m))
    stage.append(pltpu.async_copy(seg_hbm.at[sid], seg_v, sem_m))
    stage.append(pltpu.async_copy(zeros_hbm, acc_slot, sem_m))
    for c in stage:
        c.wait()

    fg = [pltpu.async_copy(tab_hbm.at[fidx_v.at[k]], frows_v.at[k], sem_f)
          for k, (_, tab_hbm, _) in enumerate(feats)]

    def h_gather(c, b):
        return pltpu.async_copy(tab_hist.at[hidx_v.at[c]], hbuf_v.at[b],
                                sem_g[b])

    def h_scatter(c, b):
        return pltpu.async_copy(hbuf_v.at[b], acc_sh.at[seg_v.at[c]],
                                sem_s[b], add=True)

    h_gather(0, 0)
    h_gather(1, 1)

    def group(j, carry):
        for b in range(2):
            c = 2 * j + b
            pltpu.make_async_copy(tab_hist.at[hidx_v.at[0]], hbuf_v.at[b],
                                  sem_g[b]).wait()
            h_scatter(c, b)

        @pl.when(j < _NCHUNK // 2 - 1)
        def _():
            for b in range(2):
                pltpu.make_async_copy(hbuf_v.at[b],
                                      acc_sh.at[seg_v.at[0]],
                                      sem_s[b]).wait()
                h_gather(2 * j + 2 + b, b)

        return carry

    lax.fori_loop(0, _NCHUNK // 2, group, 0)
    for b in range(2):
        pltpu.make_async_copy(hbuf_v.at[b], acc_sh.at[seg_v.at[0]],
                              sem_s[b]).wait()

    for c in fg:
        c.wait()
    outw = [pltpu.async_copy(frows_v.at[k], out_hbm.at[pl.ds(base, _BPW)],
                             sem_o)
            for k, (_, _, out_hbm) in enumerate(feats)]
    outw.append(pltpu.async_copy(acc_slot, out_hist.at[pl.ds(base, _BPW)],
                                 sem_o))
    for c in outw:
        c.wait()


def _mlp_body(eu, ei, ec, eh, ed, ehist, w1, b1, w2, b2, w3, b3, out):
    x = jnp.concatenate(
        [eu[...], ei[...], ec[...], eh[...], ed[...],
         ehist[...] * (1.0 / HIST)], axis=1)
    h = jax.nn.relu(jnp.dot(x, w1[...], preferred_element_type=jnp.float32)
                    + b1[...])
    h = jax.nn.relu(jnp.dot(h, w2[...], preferred_element_type=jnp.float32)
                    + b2[...])
    out[...] = jax.nn.sigmoid(
        jnp.dot(h, w3[...], preferred_element_type=jnp.float32) + b3[...])


def _mlp(feats, W1, b1, W2, b2, W3, b3):
    BB = 512
    grid = (B // BB,)
    feat_spec = pl.BlockSpec((BB, D), lambda i: (i, 0))
    full = lambda shape: pl.BlockSpec(shape, lambda i: tuple(0 for _ in shape))
    return pl.pallas_call(
        _mlp_body,
        grid=grid,
        in_specs=[feat_spec] * 6 + [
            full((ELEM, ELEM)), full((1, ELEM)),
            full((ELEM, ELEM)), full((1, ELEM)),
            full((ELEM, 1)), full((1, 1)),
        ],
        out_specs=pl.BlockSpec((BB, 1), lambda i: (i, 0)),
        out_shape=jax.ShapeDtypeStruct((B, 1), jnp.float32),
    )(*feats, W1, b1.reshape(1, ELEM), W2, b2.reshape(1, ELEM),
      W3, b3.reshape(1, 1))


def kernel(idx_user, idx_item, idx_cate, idx_hour, idx_device, idx_hist,
           tab_user, tab_item, tab_cate, tab_hour, tab_device, tab_hist,
           W1, b1, W2, b2, W3, b3):
    i32 = lambda a: a.astype(jnp.int32)
    hist_flat = i32(idx_hist).reshape(_NW, _NCHUNK, _CHUNK)
    seg_local = (jnp.arange(_BPW * HIST, dtype=jnp.int32) // HIST).reshape(
        1, _NCHUNK, _CHUNK)
    seg = seg_local + (jnp.arange(_NS, dtype=jnp.int32) * _BPW).reshape(
        _NS, 1, 1)
    zeros = jnp.zeros((_BPW, D), jnp.float32)
    feats = _sc_gather(
        i32(idx_user), i32(idx_item), i32(idx_cate), i32(idx_hour),
        i32(idx_device), hist_flat, seg, zeros,
        tab_user, tab_item, tab_cate, tab_hour, tab_device, tab_hist)
    return _mlp(feats, W1, b1, W2, b2, W3, b3)
